# baseline (speedup 1.0000x reference)
import jax
import jax.numpy as jnp
from jax import lax
from jax.experimental import pallas as pl
from jax.experimental.pallas import tpu as pltpu
from jax.experimental.pallas import tpu_sc as plsc

F = 26
CARD = 100000
D = 32
B = 16384
NC, NS, L = 2, 16, 16
NW = NC * NS
RPW = B // NW
G = 128
NG = RPW // G


def _body(xT, table, bias, out, idx_v, rows_v, bias_v, sem):
    wid = lax.axis_index("s") * NC + lax.axis_index("c")
    base = wid * RPW
    pltpu.sync_copy(bias, bias_v)

    def chunk(f, _):
        for g in range(NG):
            pltpu.sync_copy(xT.at[f, pl.ds(base + g * G, G)], idx_v.at[g])
        off = f * CARD

        def add_off(j, _):
            def add_off_col(k, _):
                idx_v[j, pl.ds(k * L, L)] = idx_v[j, pl.ds(k * L, L)] + off
                return 0
            return lax.fori_loop(0, G // L, add_off_col, 0)

        lax.fori_loop(0, NG, add_off, 0)

        cps = [
            pltpu.async_copy(table.at[idx_v.at[g]],
                             rows_v.at[pl.ds(g * G, G)], sem)
            for g in range(NG)
        ]
        for cp in cps:
            cp.wait()

        b_lo = bias_v[f, pl.ds(0, L)]
        b_hi = bias_v[f, pl.ds(L, L)]

        def add_bias(i, _):
            rows_v[i, pl.ds(0, L)] = rows_v[i, pl.ds(0, L)] + b_lo
            rows_v[i, pl.ds(L, L)] = rows_v[i, pl.ds(L, L)] + b_hi
            return 0

        lax.fori_loop(0, RPW, add_bias, 0)

        pltpu.sync_copy(rows_v, out.at[pl.ds(base, RPW), f])
        return 0

    lax.fori_loop(0, F, chunk, 0)


def kernel(x, table, bias):
    xT = x.T
    mesh = plsc.VectorSubcoreMesh(core_axis_name="c", subcore_axis_name="s")
    k = pl.kernel(
        _body,
        out_type=jax.ShapeDtypeStruct((B, F, D), jnp.float32),
        mesh=mesh,
        scratch_types=[
            pltpu.VMEM((NG, G), jnp.int32),
            pltpu.VMEM((RPW, D), jnp.float32),
            pltpu.VMEM((F, D), jnp.float32),
            pltpu.SemaphoreType.DMA,
        ],
        compiler_params=pltpu.CompilerParams(use_tc_tiling_on_sc=False),
    )
    return k(xT, table, bias)
